# baseline (speedup 1.0000x reference)
import numpy as np
import jax
import jax.numpy as jnp
from jax import lax
from jax.experimental import pallas as pl
from jax.experimental.pallas import tpu as pltpu
from jax.experimental.pallas import tpu_sc as plsc

V = 10000
C = 40000
E = 320000
G = 32
D = 128
Q = 32
LM = 8
STEPS = 4

NC = 2
NS = 16
NW = NC * NS
K = 1000
NCH = E // NW // K


def _make_edge_op(S, T, W, gather=True):
    TP = -(-T // 2048) * 2048
    TPS = TP // NS
    BZ = TPS // 4
    W16 = W // 16
    mesh = plsc.VectorSubcoreMesh(core_axis_name="c", subcore_axis_name="s",
                                  num_cores=NC)

    scratch = []
    if gather:
        scratch.append(pltpu.VMEM((NCH, K), jnp.int32))
    scratch += [
        pltpu.VMEM((NCH, K), jnp.int32),
        pltpu.VMEM((K, W), jnp.float32),
        pltpu.VMEM((K, W), jnp.float32),
        pltpu.VMEM((BZ, W), jnp.float32),
        pltpu.VMEM_SHARED((TP, W), jnp.float32),
        pltpu.SemaphoreType.DMA,
        pltpu.SemaphoreType.DMA,
    ]

    def body(*refs):
        if gather:
            (src_hbm, ia_hbm, ib_hbm, out_hbm,
             ia_v, ib_v, r0, r1, bounce, shared, g0, g1) = refs
        else:
            (ib_hbm, out_hbm,
             ib_v, r0, r1, bounce, shared, g0, g1) = refs
        c = lax.axis_index("c")
        s = lax.axis_index("s")
        wid = c * NS + s

        z16 = jnp.zeros((16,), jnp.float32)

        def zrow(i, carry):
            for w in range(W16):
                bounce[i, pl.ds(w * 16, 16)] = z16
            return carry

        lax.fori_loop(0, BZ, zrow, 0)
        for m in range(4):
            pltpu.sync_copy(bounce, shared.at[pl.ds(s * TPS + m * BZ, BZ)])

        pltpu.sync_copy(ib_hbm.at[wid], ib_v)
        if gather:
            pltpu.sync_copy(ia_hbm.at[wid], ia_v)
        else:
            o16 = jnp.ones((16,), jnp.float32)

            def orow(i, carry):
                for w in range(W16):
                    r0[i, pl.ds(w * 16, 16)] = o16
                return carry

            lax.fori_loop(0, K, orow, 0)

        plsc.subcore_barrier()

        if gather:
            pltpu.async_copy(src_hbm.at[ia_v.at[0]], r0, g0)
            pltpu.async_copy(src_hbm.at[ia_v.at[1]], r1, g1)

            def pair(j2, carry):
                j = j2 * 2
                pltpu.make_async_copy(src_hbm.at[ia_v.at[0]], r0, g0).wait()
                pltpu.sync_copy(r0, shared.at[ib_v.at[j]], add=True)

                @pl.when(j + 2 < NCH)
                def _():
                    pltpu.async_copy(src_hbm.at[ia_v.at[j + 2]], r0, g0)

                pltpu.make_async_copy(src_hbm.at[ia_v.at[1]], r1, g1).wait()
                pltpu.sync_copy(r1, shared.at[ib_v.at[j + 1]], add=True)

                @pl.when(j + 3 < NCH)
                def _():
                    pltpu.async_copy(src_hbm.at[ia_v.at[j + 3]], r1, g1)

                return carry

            lax.fori_loop(0, NCH // 2, pair, 0)
            if NCH % 2 == 1:
                pltpu.make_async_copy(src_hbm.at[ia_v.at[0]], r0, g0).wait()
                pltpu.sync_copy(r0, shared.at[ib_v.at[NCH - 1]], add=True)
        else:
            def ch(j, carry):
                pltpu.sync_copy(r0, shared.at[ib_v.at[j]], add=True)
                return carry

            lax.fori_loop(0, NCH, ch, 0)

        plsc.subcore_barrier()
        for m in range(4):
            pltpu.sync_copy(shared.at[pl.ds(s * TPS + m * BZ, BZ)], bounce)
            pltpu.sync_copy(bounce, out_hbm.at[c, pl.ds(s * TPS + m * BZ, BZ)])

    return pl.kernel(
        body,
        out_type=jax.ShapeDtypeStruct((NC, TP, W), jnp.float32),
        mesh=mesh,
        scratch_types=scratch,
        compiler_params=pltpu.CompilerParams(use_tc_tiling_on_sc=False),
    )


def _softplus(x):
    return jnp.maximum(x, 0.0) + jnp.log(1.0 + jnp.exp(-jnp.abs(x)))


def _full(shape):
    return pl.BlockSpec(shape, lambda i: (0, 0))


def _rows(b, w):
    return pl.BlockSpec((b, w), lambda i: (i, 0))


def _query_mlp(vn, w1, b1, w2, b2):
    BV = 1000

    def body(vn_r, w1_r, b1_r, w2_r, b2_r, q_r, spp_r, spn_r):
        u = jnp.maximum(vn_r[...] @ w1_r[...] + b1_r[...], 0.0)
        q = u @ w2_r[...] + b2_r[...]
        sp = _softplus(q)
        q_r[...] = q
        spp_r[...] = sp
        spn_r[...] = sp - q

    return pl.pallas_call(
        body,
        grid=(V // BV,),
        in_specs=[_rows(BV, 132), _full((132, 38)), _full((1, 38)),
                  _full((38, 32)), _full((1, 32))],
        out_specs=[_rows(BV, 32)] * 3,
        out_shape=[jax.ShapeDtypeStruct((V, 32), jnp.float32)] * 3,
    )(vn, w1, b1, w2, b2)


def _clause_a(cs, cv, gc, w1, b1, w2, b2):
    BC = 1000

    def body(cs_r, cv_r, g_r, w1_r, b1_r, w2_r, b2_r,
             e1_r, e2_r, v1_r, v2_r, x_r, s1_r, s2_r):
        i = pl.program_id(0)
        expcv = jnp.exp(-cv_r[...])
        u = jnp.concatenate([cs_r[...], 4.0 * expcv], axis=1)
        h = jnp.maximum(u @ w1_r[...] + b1_r[...], 0.0)
        dd = h @ w2_r[...] + b2_r[...]
        vla = dd[:, :Q]
        x = dd[:, Q:]
        e1_r[...] = expcv[:, :16]
        e2_r[...] = expcv[:, 16:]
        v1_r[...] = vla[:, :16]
        v2_r[...] = vla[:, 16:]
        x_r[...] = x
        m2 = jnp.broadcast_to(jnp.mean(x * x, axis=1, keepdims=True), (BC, 8))
        g = g_r[...]
        ds1 = lax.dot_general(g, x, (((0,), (0,)), ((), ())),
                              preferred_element_type=jnp.float32)
        ds2 = lax.dot_general(g, m2, (((0,), (0,)), ((), ())),
                              preferred_element_type=jnp.float32)

        @pl.when(i == 0)
        def _():
            s1_r[...] = jnp.zeros_like(s1_r)
            s2_r[...] = jnp.zeros_like(s2_r)

        s1_r[...] += ds1
        s2_r[...] += ds2

    return pl.pallas_call(
        body,
        grid=(C // BC,),
        in_specs=[_rows(BC, 128), _rows(BC, 32), _rows(BC, 32),
                  _full((160, 204)), _full((1, 204)),
                  _full((204, 160)), _full((1, 160))],
        out_specs=[_rows(BC, 16), _rows(BC, 16), _rows(BC, 16),
                   _rows(BC, 16), _rows(BC, 128),
                   _full((G, 128)), _full((G, 8))],
        out_shape=[jax.ShapeDtypeStruct((C, 16), jnp.float32),
                   jax.ShapeDtypeStruct((C, 16), jnp.float32),
                   jax.ShapeDtypeStruct((C, 16), jnp.float32),
                   jax.ShapeDtypeStruct((C, 16), jnp.float32),
                   jax.ShapeDtypeStruct((C, 128), jnp.float32),
                   jax.ShapeDtypeStruct((G, 128), jnp.float32),
                   jax.ShapeDtypeStruct((G, 8), jnp.float32)],
    )(cs, cv, gc, w1, b1, w2, b2)


def _pairnorm_body(x, g, s1, s2, cnt):
    c1 = cnt[:, :1]
    mean = s1 / c1
    m2g = s2[:, :1] / c1
    var = m2g - jnp.mean(mean * mean, axis=1, keepdims=True)
    scale = lax.rsqrt(var + 1e-6)
    meanb = g @ mean
    scaleb = g @ scale
    return (x - meanb) * scaleb


def _clause_b(x, cs_old, gc, s1, s2, cnt):
    BC = 1000

    def body(x_r, cs_r, g_r, s1_r, s2_r, cnt_r, out_r):
        pn = _pairnorm_body(x_r[...], g_r[...], s1_r[...], s2_r[...],
                            cnt_r[...])
        out_r[...] = pn + 0.1 * cs_r[...]

    return pl.pallas_call(
        body,
        grid=(C // BC,),
        in_specs=[_rows(BC, 128), _rows(BC, 128), _rows(BC, 32),
                  _full((G, 128)), _full((G, 8)), _full((G, 8))],
        out_specs=[_rows(BC, 128)],
        out_shape=[jax.ShapeDtypeStruct((C, 128), jnp.float32)],
    )(x, cs_old, gc, s1, s2, cnt)[0]


def _update_a(q, te_p, te_n, tv_p, tv_n, v, vdw8, dwp8, dwn8, gv,
              w1, b1, w2, b2, w3, b3):
    BV = 1000

    def body(q_r, tep_r, ten_r, tvp_r, tvn_r, v_r, vdw_r, dwp_r, dwn_r, g_r,
             w1_r, b1_r, w2_r, b2_r, w3_r, b3_r, x_r, s1_r, s2_r):
        i = pl.program_id(0)
        qv = q_r[...]
        sig = 1.0 / (1.0 + jnp.exp(-qv))
        grad = (-sig * tep_r[...] + (1.0 - sig) * ten_r[...]) * vdw_r[:, :1]
        lp = tvp_r[...] * dwp_r[:, :1]
        ln = tvn_r[...] * dwn_r[:, :1]
        u = jnp.concatenate([grad, v_r[...], lp, ln], axis=1)
        h1 = jnp.maximum(u @ w1_r[...] + b1_r[...], 0.0)
        h2 = jnp.maximum(h1 @ w2_r[...] + b2_r[...], 0.0)
        x = h2 @ w3_r[...] + b3_r[...]
        x_r[...] = x
        m2 = jnp.broadcast_to(jnp.mean(x * x, axis=1, keepdims=True), (BV, 8))
        g = g_r[...]
        ds1 = lax.dot_general(g, x, (((0,), (0,)), ((), ())),
                              preferred_element_type=jnp.float32)
        ds2 = lax.dot_general(g, m2, (((0,), (0,)), ((), ())),
                              preferred_element_type=jnp.float32)

        @pl.when(i == 0)
        def _():
            s1_r[...] = jnp.zeros_like(s1_r)
            s2_r[...] = jnp.zeros_like(s2_r)

        s1_r[...] += ds1
        s2_r[...] += ds2

    return pl.pallas_call(
        body,
        grid=(V // BV,),
        in_specs=[_rows(BV, 32), _rows(BV, 32), _rows(BV, 32),
                  _rows(BV, 32), _rows(BV, 32),
                  _rows(BV, 128), _rows(BV, 8), _rows(BV, 8), _rows(BV, 8),
                  _rows(BV, 32),
                  _full((224, 230)), _full((1, 230)),
                  _full((230, 230)), _full((1, 230)),
                  _full((230, 128)), _full((1, 128))],
        out_specs=[_rows(BV, 128), _full((G, 128)), _full((G, 8))],
        out_shape=[jax.ShapeDtypeStruct((V, 128), jnp.float32),
                   jax.ShapeDtypeStruct((G, 128), jnp.float32),
                   jax.ShapeDtypeStruct((G, 8), jnp.float32)],
    )(q, te_p, te_n, tv_p, tv_n, v, vdw8, dwp8, dwn8, gv,
      w1, b1, w2, b2, w3, b3)


def _update_b(x, v_old, gv, s1, s2, cnt, wo1, bo1, wo2, bo2):
    BV = 1000

    def body(x_r, v_r, g_r, s1_r, s2_r, cnt_r, wo1_r, bo1_r, wo2_r, bo2_r,
             vn_r, tp_r, tn_r):
        pn = _pairnorm_body(x_r[...], g_r[...], s1_r[...], s2_r[...],
                            cnt_r[...])
        vnew = pn + 0.1 * v_r[...]
        vn_r[...] = vnew
        h = jnp.maximum(vnew @ wo1_r[...] + bo1_r[...], 0.0)
        lg = h @ wo2_r[...] + bo2_r[...]
        sp = _softplus(lg)
        hard = (lg > 0).astype(jnp.float32)
        tp_r[...] = jnp.concatenate([sp, hard], axis=1)
        tn_r[...] = jnp.concatenate([sp - lg, 1.0 - hard], axis=1)

    return pl.pallas_call(
        body,
        grid=(V // BV,),
        in_specs=[_rows(BV, 128), _rows(BV, 128), _rows(BV, 32),
                  _full((G, 128)), _full((G, 8)), _full((G, 8)),
                  _full((128, 128)), _full((1, 128)),
                  _full((128, 8)), _full((1, 8))],
        out_specs=[_rows(BV, 128), _rows(BV, 16), _rows(BV, 16)],
        out_shape=[jax.ShapeDtypeStruct((V, 128), jnp.float32),
                   jax.ShapeDtypeStruct((V, 16), jnp.float32),
                   jax.ShapeDtypeStruct((V, 16), jnp.float32)],
    )(x, v_old, gv, s1, s2, cnt, wo1, bo1, wo2, bo2)


def _loss_reduce(acc, gc):
    BC = 1000

    def body(a_r, g_r, pg_r):
        i = pl.program_id(0)
        a = a_r[...]
        cv8 = jnp.exp(-a[:, :LM])
        pcl = cv8 * (-jnp.log(1.0 - cv8 + 1e-6))
        hard = jnp.minimum(a[:, LM:], 1.0)
        both = jnp.concatenate([pcl, hard], axis=1)
        d = lax.dot_general(g_r[...], both, (((0,), (0,)), ((), ())),
                            preferred_element_type=jnp.float32)

        @pl.when(i == 0)
        def _():
            pg_r[...] = jnp.zeros_like(pg_r)

        pg_r[...] += d

    return pl.pallas_call(
        body,
        grid=(C // BC,),
        in_specs=[_rows(BC, 16), _rows(BC, 32)],
        out_specs=[_full((G, 16))],
        out_shape=[jax.ShapeDtypeStruct((G, 16), jnp.float32)],
    )(acc, gc)[0]


def kernel(params, variable_indices, clause_indices, polarity,
           variable_batch, clause_batch):
    (wq1, bq1), (wq2, bq2) = params['variables_query']
    (wc1, bc1), (wc2, bc2) = params['clause_mlp']
    (wu1, bu1), (wu2, bu2), (wu3, bu3) = params['update_gate']
    (wo1, bo1), (wo2, bo2) = params['variables_output']
    bq1 = bq1.reshape(1, -1); bq2 = bq2.reshape(1, -1)
    bc1 = bc1.reshape(1, -1); bc2 = bc2.reshape(1, -1)
    bu1 = bu1.reshape(1, -1); bu2 = bu2.reshape(1, -1)
    bu3 = bu3.reshape(1, -1)
    bo1 = bo1.reshape(1, -1); bo2 = bo2.reshape(1, -1)

    vi = variable_indices.astype(jnp.int32)
    ci = clause_indices.astype(jnp.int32)
    lit = vi + jnp.where(polarity > 0, 0, V).astype(jnp.int32)
    lit2 = lit.reshape(NW, NCH, K)
    ci2 = ci.reshape(NW, NCH, K)

    gv = (variable_batch[:, None] == jnp.arange(G)[None, :]).astype(jnp.float32)
    gc = (clause_batch[:, None] == jnp.arange(G)[None, :]).astype(jnp.float32)
    var_counts = jnp.maximum(jnp.sum(gv, axis=0), 1.0)[:, None]
    cl_counts = jnp.maximum(jnp.sum(gc, axis=0), 1.0)[:, None]
    cnt_v8 = jnp.broadcast_to(var_counts, (G, 8))
    cnt_c8 = jnp.broadcast_to(cl_counts, (G, 8))

    edge_lc16 = _make_edge_op(2 * V, C, 16)
    edge_cl16 = _make_edge_op(C, 2 * V, 16)

    ones_c16 = jnp.ones((C, 16), jnp.float32)
    degp = edge_cl16(ones_c16, ci2, lit2)
    deg = (degp[0] + degp[1])[:2 * V, :1]
    degree_weight = lax.rsqrt(jnp.maximum(deg, 1.0))
    vdw = 4.0 * lax.rsqrt(jnp.maximum(deg[:V] + deg[V], 1.0))
    vdw8 = jnp.broadcast_to(vdw, (V, 8))
    dwp8 = jnp.broadcast_to(degree_weight[:V], (V, 8))
    dwn8 = jnp.broadcast_to(degree_weight[V:], (V, 8))

    variables = jnp.ones((V, D), jnp.float32)
    clause_state = jnp.ones((C, D), jnp.float32)
    noise_key = jax.random.key(42)
    costs = jnp.square(jnp.arange(1, LM + 1, dtype=jnp.float32))

    losses = []
    solveds = []
    for step in range(STEPS):
        noise = jax.random.normal(jax.random.fold_in(noise_key, step),
                                  (V, 4), jnp.float32)
        vn = jnp.concatenate([variables, noise], axis=1)
        q, sp_p, sp_n = _query_mlp(vn, wq1, bq1, wq2, bq2)
        lit_tab = jnp.concatenate([sp_p, sp_n], axis=0)

        cvh1 = edge_lc16(lit_tab[:, :16], lit2, ci2)
        cvh2 = edge_lc16(lit_tab[:, 16:], lit2, ci2)
        cv = jnp.concatenate([(cvh1[0] + cvh1[1])[:C],
                              (cvh2[0] + cvh2[1])[:C]], axis=1)

        e1, e2, v1, v2, x_c, s1c, s2c = _clause_a(clause_state, cv, gc,
                                                  wc1, bc1, wc2, bc2)
        clause_state = _clause_b(x_c, clause_state, gc, s1c, s2c, cnt_c8)

        teh1 = edge_cl16(e1, ci2, lit2)
        teh2 = edge_cl16(e2, ci2, lit2)
        tvh1 = edge_cl16(v1, ci2, lit2)
        tvh2 = edge_cl16(v2, ci2, lit2)
        t_e = jnp.concatenate([(teh1[0] + teh1[1])[:2 * V],
                               (teh2[0] + teh2[1])[:2 * V]], axis=1)
        t_v = jnp.concatenate([(tvh1[0] + tvh1[1])[:2 * V],
                               (tvh2[0] + tvh2[1])[:2 * V]], axis=1)

        x_v, s1v, s2v = _update_a(q, t_e[:V], t_e[V:], t_v[:V], t_v[V:],
                                  variables, vdw8, dwp8, dwn8, gv,
                                  wu1, bu1, wu2, bu2, wu3, bu3)
        variables, tabp, tabn = _update_b(x_v, variables, gv, s1v, s2v,
                                          cnt_v8, wo1, bo1, wo2, bo2)
        tab16 = jnp.concatenate([tabp, tabn], axis=0)

        accp = edge_lc16(tab16, lit2, ci2)
        acc = (accp[0] + accp[1])[:C]

        pg = _loss_reduce(acc, gc)
        pgl = jnp.sqrt(pg[:, :LM] + 1e-6) - np.sqrt(1e-6)
        sorted_loss = -jnp.sort(-pgl, axis=-1)
        losses.append(jnp.sum(sorted_loss * costs) / jnp.sum(costs))
        gh = (pg[:, LM:] / cl_counts) > (1.0 - 1e-6)
        solveds.append(jnp.any(gh, axis=1))

    loss = jnp.sum(jnp.stack(losses)) / STEPS
    return loss, jnp.stack(solveds)


# speedup vs baseline: 1.3910x; 1.3910x over previous
import numpy as np
import jax
import jax.numpy as jnp
from jax import lax
from jax.experimental import pallas as pl
from jax.experimental.pallas import tpu as pltpu
from jax.experimental.pallas import tpu_sc as plsc

V = 10000
C = 40000
E = 320000
G = 32
D = 128
Q = 32
LM = 8
STEPS = 4

NC = 2
NS = 16
NW = NC * NS
K = 1000
NCH = E // NW // K


def _make_edge_op(S, T, W, gather=True):
    TP = -(-T // 2048) * 2048
    TPS = TP // NS
    BZ = TPS // 4
    W16 = W // 16
    mesh = plsc.VectorSubcoreMesh(core_axis_name="c", subcore_axis_name="s",
                                  num_cores=NC)

    scratch = []
    if gather:
        scratch.append(pltpu.VMEM((NCH, K), jnp.int32))
    scratch += [
        pltpu.VMEM((NCH, K), jnp.int32),
        pltpu.VMEM((K, W), jnp.float32),
        pltpu.VMEM((K, W), jnp.float32),
        pltpu.VMEM((BZ, W), jnp.float32),
        pltpu.VMEM_SHARED((TP, W), jnp.float32),
        pltpu.SemaphoreType.DMA,
        pltpu.SemaphoreType.DMA,
    ]

    def body(*refs):
        if gather:
            (src_hbm, ia_hbm, ib_hbm, out_hbm,
             ia_v, ib_v, r0, r1, bounce, shared, g0, g1) = refs
        else:
            (ib_hbm, out_hbm,
             ib_v, r0, r1, bounce, shared, g0, g1) = refs
        c = lax.axis_index("c")
        s = lax.axis_index("s")
        wid = c * NS + s

        z16 = jnp.zeros((16,), jnp.float32)

        def zrow(i, carry):
            for w in range(W16):
                bounce[i, pl.ds(w * 16, 16)] = z16
            return carry

        lax.fori_loop(0, BZ, zrow, 0)
        for m in range(4):
            pltpu.sync_copy(bounce, shared.at[pl.ds(s * TPS + m * BZ, BZ)])

        pltpu.sync_copy(ib_hbm.at[wid], ib_v)
        if gather:
            pltpu.sync_copy(ia_hbm.at[wid], ia_v)
        else:
            o16 = jnp.ones((16,), jnp.float32)

            def orow(i, carry):
                for w in range(W16):
                    r0[i, pl.ds(w * 16, 16)] = o16
                return carry

            lax.fori_loop(0, K, orow, 0)

        plsc.subcore_barrier()

        if gather:
            pltpu.async_copy(src_hbm.at[ia_v.at[0]], r0, g0)
            pltpu.async_copy(src_hbm.at[ia_v.at[1]], r1, g1)

            def pair(j2, carry):
                j = j2 * 2
                pltpu.make_async_copy(src_hbm.at[ia_v.at[0]], r0, g0).wait()
                pltpu.sync_copy(r0, shared.at[ib_v.at[j]], add=True)

                @pl.when(j + 2 < NCH)
                def _():
                    pltpu.async_copy(src_hbm.at[ia_v.at[j + 2]], r0, g0)

                pltpu.make_async_copy(src_hbm.at[ia_v.at[1]], r1, g1).wait()
                pltpu.sync_copy(r1, shared.at[ib_v.at[j + 1]], add=True)

                @pl.when(j + 3 < NCH)
                def _():
                    pltpu.async_copy(src_hbm.at[ia_v.at[j + 3]], r1, g1)

                return carry

            lax.fori_loop(0, NCH // 2, pair, 0)
            if NCH % 2 == 1:
                pltpu.make_async_copy(src_hbm.at[ia_v.at[0]], r0, g0).wait()
                pltpu.sync_copy(r0, shared.at[ib_v.at[NCH - 1]], add=True)
        else:
            def ch(j, carry):
                pltpu.sync_copy(r0, shared.at[ib_v.at[j]], add=True)
                return carry

            lax.fori_loop(0, NCH, ch, 0)

        plsc.subcore_barrier()
        for m in range(4):
            pltpu.sync_copy(shared.at[pl.ds(s * TPS + m * BZ, BZ)], bounce)
            pltpu.sync_copy(bounce, out_hbm.at[c, pl.ds(s * TPS + m * BZ, BZ)])

    return pl.kernel(
        body,
        out_type=jax.ShapeDtypeStruct((NC, TP, W), jnp.float32),
        mesh=mesh,
        scratch_types=scratch,
        compiler_params=pltpu.CompilerParams(use_tc_tiling_on_sc=False),
    )


def _softplus(x):
    return jnp.maximum(x, 0.0) + jnp.log(1.0 + jnp.exp(-jnp.abs(x)))


TPC = -(-C // 2048) * 2048
TPL = -(-2 * V // 2048) * 2048


def _part(b, w, core, off=0):
    return pl.BlockSpec((1, b, w), lambda i, c=core, o=off: (c, i + o, 0))


def _full(shape):
    return pl.BlockSpec(shape, lambda i: (0, 0))


def _rows(b, w):
    return pl.BlockSpec((b, w), lambda i: (i, 0))


def _query_mlp(v, noise, w1v, w1n, b1, w2, b2):
    BV = 1000

    def body(v_r, n_r, w1v_r, w1n_r, b1_r, w2_r, b2_r, q_r, t1_r, t2_r):
        i = pl.program_id(0)
        u = jnp.maximum(v_r[...] @ w1v_r[...] + n_r[...] @ w1n_r[...]
                        + b1_r[...], 0.0)
        q = u @ w2_r[...] + b2_r[...]
        sp = _softplus(q)
        tab = jnp.where(i >= V // BV, sp - q, sp)
        q_r[...] = q
        t1_r[...] = tab[:, :16]
        t2_r[...] = tab[:, 16:]

    half = lambda i: (i % (V // BV), 0)
    return pl.pallas_call(
        body,
        grid=(2 * V // BV,),
        in_specs=[pl.BlockSpec((BV, 128), half), pl.BlockSpec((BV, 4), half),
                  _full((128, 38)), _full((4, 38)), _full((1, 38)),
                  _full((38, 32)), _full((1, 32))],
        out_specs=[pl.BlockSpec((BV, 32), half),
                   _rows(BV, 16), _rows(BV, 16)],
        out_shape=[jax.ShapeDtypeStruct((V, 32), jnp.float32),
                   jax.ShapeDtypeStruct((2 * V, 16), jnp.float32),
                   jax.ShapeDtypeStruct((2 * V, 16), jnp.float32)],
    )(v, noise, w1v, w1n, b1, w2, b2)


def _clause_a(cs, cvh1, cvh2, gc, w1, b1, w2, b2):
    BC = 1000

    def body(cs_r, h1a_r, h1b_r, h2a_r, h2b_r, g_r, w1_r, b1_r, w2_r, b2_r,
             e1_r, e2_r, v1_r, v2_r, x_r, s1_r, s2_r):
        i = pl.program_id(0)
        cv = jnp.concatenate([h1a_r[0] + h1b_r[0], h2a_r[0] + h2b_r[0]],
                             axis=1)
        expcv = jnp.exp(-cv)
        u = jnp.concatenate([cs_r[...], 4.0 * expcv], axis=1)
        h = jnp.maximum(u @ w1_r[...] + b1_r[...], 0.0)
        dd = h @ w2_r[...] + b2_r[...]
        vla = dd[:, :Q]
        x = dd[:, Q:]
        e1_r[...] = expcv[:, :16]
        e2_r[...] = expcv[:, 16:]
        v1_r[...] = vla[:, :16]
        v2_r[...] = vla[:, 16:]
        x_r[...] = x
        m2 = jnp.broadcast_to(jnp.mean(x * x, axis=1, keepdims=True), (BC, 8))
        g = g_r[...]
        ds1 = lax.dot_general(g, x, (((0,), (0,)), ((), ())),
                              preferred_element_type=jnp.float32)
        ds2 = lax.dot_general(g, m2, (((0,), (0,)), ((), ())),
                              preferred_element_type=jnp.float32)

        @pl.when(i == 0)
        def _():
            s1_r[...] = jnp.zeros_like(s1_r)
            s2_r[...] = jnp.zeros_like(s2_r)

        s1_r[...] += ds1
        s2_r[...] += ds2

    return pl.pallas_call(
        body,
        grid=(C // BC,),
        in_specs=[_rows(BC, 128),
                  _part(BC, 16, 0), _part(BC, 16, 1),
                  _part(BC, 16, 0), _part(BC, 16, 1),
                  _rows(BC, 32),
                  _full((160, 204)), _full((1, 204)),
                  _full((204, 160)), _full((1, 160))],
        out_specs=[_rows(BC, 16), _rows(BC, 16), _rows(BC, 16),
                   _rows(BC, 16), _rows(BC, 128),
                   _full((G, 128)), _full((G, 8))],
        out_shape=[jax.ShapeDtypeStruct((C, 16), jnp.float32),
                   jax.ShapeDtypeStruct((C, 16), jnp.float32),
                   jax.ShapeDtypeStruct((C, 16), jnp.float32),
                   jax.ShapeDtypeStruct((C, 16), jnp.float32),
                   jax.ShapeDtypeStruct((C, 128), jnp.float32),
                   jax.ShapeDtypeStruct((G, 128), jnp.float32),
                   jax.ShapeDtypeStruct((G, 8), jnp.float32)],
    )(cs, cvh1, cvh1, cvh2, cvh2, gc, w1, b1, w2, b2)


def _pairnorm_body(x, g, s1, s2, cnt):
    c1 = cnt[:, :1]
    mean = s1 / c1
    m2g = s2[:, :1] / c1
    var = m2g - jnp.mean(mean * mean, axis=1, keepdims=True)
    scale = lax.rsqrt(var + 1e-6)
    meanb = g @ mean
    scaleb = g @ scale
    return (x - meanb) * scaleb


def _clause_b(x, cs_old, gc, s1, s2, cnt):
    BC = 1000

    def body(x_r, cs_r, g_r, s1_r, s2_r, cnt_r, out_r):
        pn = _pairnorm_body(x_r[...], g_r[...], s1_r[...], s2_r[...],
                            cnt_r[...])
        out_r[...] = pn + 0.1 * cs_r[...]

    return pl.pallas_call(
        body,
        grid=(C // BC,),
        in_specs=[_rows(BC, 128), _rows(BC, 128), _rows(BC, 32),
                  _full((G, 128)), _full((G, 8)), _full((G, 8))],
        out_specs=[_rows(BC, 128)],
        out_shape=[jax.ShapeDtypeStruct((C, 128), jnp.float32)],
    )(x, cs_old, gc, s1, s2, cnt)[0]


def _update_a(q, teh1, teh2, tvh1, tvh2, v, vdw8, dwp8, dwn8, gv,
              w1, b1, w2, b2, w3, b3):
    BV = 1000
    NB = V // BV

    def body(q_r, e1pa, e1pb, e2pa, e2pb, e1na, e1nb, e2na, e2nb,
             v1pa, v1pb, v2pa, v2pb, v1na, v1nb, v2na, v2nb,
             v_r, vdw_r, dwp_r, dwn_r, g_r,
             w1_r, b1_r, w2_r, b2_r, w3_r, b3_r, x_r, s1_r, s2_r):
        i = pl.program_id(0)
        qv = q_r[...]
        sig = 1.0 / (1.0 + jnp.exp(-qv))
        te_p = jnp.concatenate([e1pa[0] + e1pb[0], e2pa[0] + e2pb[0]], axis=1)
        te_n = jnp.concatenate([e1na[0] + e1nb[0], e2na[0] + e2nb[0]], axis=1)
        tv_p = jnp.concatenate([v1pa[0] + v1pb[0], v2pa[0] + v2pb[0]], axis=1)
        tv_n = jnp.concatenate([v1na[0] + v1nb[0], v2na[0] + v2nb[0]], axis=1)
        grad = (-sig * te_p + (1.0 - sig) * te_n) * vdw_r[:, :1]
        lp = tv_p * dwp_r[:, :1]
        ln = tv_n * dwn_r[:, :1]
        u = jnp.concatenate([grad, v_r[...], lp, ln], axis=1)
        h1 = jnp.maximum(u @ w1_r[...] + b1_r[...], 0.0)
        h2 = jnp.maximum(h1 @ w2_r[...] + b2_r[...], 0.0)
        x = h2 @ w3_r[...] + b3_r[...]
        x_r[...] = x
        m2 = jnp.broadcast_to(jnp.mean(x * x, axis=1, keepdims=True), (BV, 8))
        g = g_r[...]
        ds1 = lax.dot_general(g, x, (((0,), (0,)), ((), ())),
                              preferred_element_type=jnp.float32)
        ds2 = lax.dot_general(g, m2, (((0,), (0,)), ((), ())),
                              preferred_element_type=jnp.float32)

        @pl.when(i == 0)
        def _():
            s1_r[...] = jnp.zeros_like(s1_r)
            s2_r[...] = jnp.zeros_like(s2_r)

        s1_r[...] += ds1
        s2_r[...] += ds2

    return pl.pallas_call(
        body,
        grid=(V // BV,),
        in_specs=[_rows(BV, 32),
                  _part(BV, 16, 0), _part(BV, 16, 1),
                  _part(BV, 16, 0), _part(BV, 16, 1),
                  _part(BV, 16, 0, NB), _part(BV, 16, 1, NB),
                  _part(BV, 16, 0, NB), _part(BV, 16, 1, NB),
                  _part(BV, 16, 0), _part(BV, 16, 1),
                  _part(BV, 16, 0), _part(BV, 16, 1),
                  _part(BV, 16, 0, NB), _part(BV, 16, 1, NB),
                  _part(BV, 16, 0, NB), _part(BV, 16, 1, NB),
                  _rows(BV, 128), _rows(BV, 8), _rows(BV, 8), _rows(BV, 8),
                  _rows(BV, 32),
                  _full((224, 230)), _full((1, 230)),
                  _full((230, 230)), _full((1, 230)),
                  _full((230, 128)), _full((1, 128))],
        out_specs=[_rows(BV, 128), _full((G, 128)), _full((G, 8))],
        out_shape=[jax.ShapeDtypeStruct((V, 128), jnp.float32),
                   jax.ShapeDtypeStruct((G, 128), jnp.float32),
                   jax.ShapeDtypeStruct((G, 8), jnp.float32)],
    )(q, teh1, teh1, teh2, teh2, teh1, teh1, teh2, teh2,
      tvh1, tvh1, tvh2, tvh2, tvh1, tvh1, tvh2, tvh2,
      v, vdw8, dwp8, dwn8, gv, w1, b1, w2, b2, w3, b3)


def _update_b(x, v_old, gv, s1, s2, cnt, wo1, bo1, wo2, bo2):
    BV = 1000

    def body(x_r, v_r, g_r, s1_r, s2_r, cnt_r, wo1_r, bo1_r, wo2_r, bo2_r,
             vn_r, tab_r):
        i = pl.program_id(0)
        pn = _pairnorm_body(x_r[...], g_r[...], s1_r[...], s2_r[...],
                            cnt_r[...])
        vnew = pn + 0.1 * v_r[...]
        vn_r[...] = vnew
        h = jnp.maximum(vnew @ wo1_r[...] + bo1_r[...], 0.0)
        lg = h @ wo2_r[...] + bo2_r[...]
        sp = _softplus(lg)
        hard = (lg > 0).astype(jnp.float32)
        neg = i >= V // BV
        tab_r[...] = jnp.where(neg,
                               jnp.concatenate([sp - lg, 1.0 - hard], axis=1),
                               jnp.concatenate([sp, hard], axis=1))

    half = lambda i: (i % (V // BV), 0)
    return pl.pallas_call(
        body,
        grid=(2 * V // BV,),
        in_specs=[pl.BlockSpec((BV, 128), half), pl.BlockSpec((BV, 128), half),
                  pl.BlockSpec((BV, 32), half),
                  _full((G, 128)), _full((G, 8)), _full((G, 8)),
                  _full((128, 128)), _full((1, 128)),
                  _full((128, 8)), _full((1, 8))],
        out_specs=[pl.BlockSpec((BV, 128), half), _rows(BV, 16)],
        out_shape=[jax.ShapeDtypeStruct((V, 128), jnp.float32),
                   jax.ShapeDtypeStruct((2 * V, 16), jnp.float32)],
    )(x, v_old, gv, s1, s2, cnt, wo1, bo1, wo2, bo2)


def _loss_reduce(accp, gc):
    BC = 1000

    def body(aa_r, ab_r, g_r, pg_r):
        i = pl.program_id(0)
        a = aa_r[0] + ab_r[0]
        cv8 = jnp.exp(-a[:, :LM])
        pcl = cv8 * (-jnp.log(1.0 - cv8 + 1e-6))
        hard = jnp.minimum(a[:, LM:], 1.0)
        both = jnp.concatenate([pcl, hard], axis=1)
        d = lax.dot_general(g_r[...], both, (((0,), (0,)), ((), ())),
                            preferred_element_type=jnp.float32)

        @pl.when(i == 0)
        def _():
            pg_r[...] = jnp.zeros_like(pg_r)

        pg_r[...] += d

    return pl.pallas_call(
        body,
        grid=(C // BC,),
        in_specs=[_part(BC, 16, 0), _part(BC, 16, 1), _rows(BC, 32)],
        out_specs=[_full((G, 16))],
        out_shape=[jax.ShapeDtypeStruct((G, 16), jnp.float32)],
    )(accp, accp, gc)[0]


def kernel(params, variable_indices, clause_indices, polarity,
           variable_batch, clause_batch):
    (wq1, bq1), (wq2, bq2) = params['variables_query']
    (wc1, bc1), (wc2, bc2) = params['clause_mlp']
    (wu1, bu1), (wu2, bu2), (wu3, bu3) = params['update_gate']
    (wo1, bo1), (wo2, bo2) = params['variables_output']
    bq1 = bq1.reshape(1, -1); bq2 = bq2.reshape(1, -1)
    bc1 = bc1.reshape(1, -1); bc2 = bc2.reshape(1, -1)
    bu1 = bu1.reshape(1, -1); bu2 = bu2.reshape(1, -1)
    bu3 = bu3.reshape(1, -1)
    bo1 = bo1.reshape(1, -1); bo2 = bo2.reshape(1, -1)

    vi = variable_indices.astype(jnp.int32)
    ci = clause_indices.astype(jnp.int32)
    lit = vi + jnp.where(polarity > 0, 0, V).astype(jnp.int32)
    lit2 = lit.reshape(NW, NCH, K)
    ci2 = ci.reshape(NW, NCH, K)

    gv = (variable_batch[:, None] == jnp.arange(G)[None, :]).astype(jnp.float32)
    gc = (clause_batch[:, None] == jnp.arange(G)[None, :]).astype(jnp.float32)
    var_counts = jnp.maximum(jnp.sum(gv, axis=0), 1.0)[:, None]
    cl_counts = jnp.maximum(jnp.sum(gc, axis=0), 1.0)[:, None]
    cnt_v8 = jnp.broadcast_to(var_counts, (G, 8))
    cnt_c8 = jnp.broadcast_to(cl_counts, (G, 8))

    edge_lc16 = _make_edge_op(2 * V, C, 16)
    edge_cl16 = _make_edge_op(C, 2 * V, 16)

    ones_c16 = jnp.ones((C, 16), jnp.float32)
    degp = edge_cl16(ones_c16, ci2, lit2)
    deg = (degp[0] + degp[1])[:2 * V, :1]
    degree_weight = lax.rsqrt(jnp.maximum(deg, 1.0))
    vdw = 4.0 * lax.rsqrt(jnp.maximum(deg[:V] + deg[V], 1.0))
    vdw8 = jnp.broadcast_to(vdw, (V, 8))
    dwp8 = jnp.broadcast_to(degree_weight[:V], (V, 8))
    dwn8 = jnp.broadcast_to(degree_weight[V:], (V, 8))

    variables = jnp.ones((V, D), jnp.float32)
    clause_state = jnp.ones((C, D), jnp.float32)
    noise_key = jax.random.key(42)
    costs = jnp.square(jnp.arange(1, LM + 1, dtype=jnp.float32))

    wq1v = wq1[:D]
    wq1n = wq1[D:]

    losses = []
    solveds = []
    for step in range(STEPS):
        noise = jax.random.normal(jax.random.fold_in(noise_key, step),
                                  (V, 4), jnp.float32)
        q, lt1, lt2 = _query_mlp(variables, noise, wq1v, wq1n, bq1, wq2, bq2)

        cvh1 = edge_lc16(lt1, lit2, ci2)
        cvh2 = edge_lc16(lt2, lit2, ci2)

        e1, e2, v1, v2, x_c, s1c, s2c = _clause_a(clause_state, cvh1, cvh2,
                                                  gc, wc1, bc1, wc2, bc2)
        clause_state = _clause_b(x_c, clause_state, gc, s1c, s2c, cnt_c8)

        teh1 = edge_cl16(e1, ci2, lit2)
        teh2 = edge_cl16(e2, ci2, lit2)
        tvh1 = edge_cl16(v1, ci2, lit2)
        tvh2 = edge_cl16(v2, ci2, lit2)

        x_v, s1v, s2v = _update_a(q, teh1, teh2, tvh1, tvh2,
                                  variables, vdw8, dwp8, dwn8, gv,
                                  wu1, bu1, wu2, bu2, wu3, bu3)
        variables, tab16 = _update_b(x_v, variables, gv, s1v, s2v,
                                     cnt_v8, wo1, bo1, wo2, bo2)

        accp = edge_lc16(tab16, lit2, ci2)

        pg = _loss_reduce(accp, gc)
        pgl = jnp.sqrt(pg[:, :LM] + 1e-6) - np.sqrt(1e-6)
        sorted_loss = -jnp.sort(-pgl, axis=-1)
        losses.append(jnp.sum(sorted_loss * costs) / jnp.sum(costs))
        gh = (pg[:, LM:] / cl_counts) > (1.0 - 1e-6)
        solveds.append(jnp.any(gh, axis=1))

    loss = jnp.sum(jnp.stack(losses)) / STEPS
    return loss, jnp.stack(solveds)


# speedup vs baseline: 1.4044x; 1.0096x over previous
import numpy as np
import jax
import jax.numpy as jnp
from jax import lax
from jax.experimental import pallas as pl
from jax.experimental.pallas import tpu as pltpu
from jax.experimental.pallas import tpu_sc as plsc

V = 10000
C = 40000
E = 320000
G = 32
D = 128
Q = 32
LM = 8
STEPS = 4

NC = 2
NS = 16
NW = NC * NS
K = 1000
NCH = E // NW // K


def _make_edge_op(S, T, W, gather=True):
    TP = -(-T // 2048) * 2048
    TPS = TP // NS
    BZ = TPS // 4
    W16 = W // 16
    mesh = plsc.VectorSubcoreMesh(core_axis_name="c", subcore_axis_name="s",
                                  num_cores=NC)

    scratch = []
    if gather:
        scratch.append(pltpu.VMEM((NCH, K), jnp.int32))
    scratch += [
        pltpu.VMEM((NCH, K), jnp.int32),
        pltpu.VMEM((K, W), jnp.float32),
        pltpu.VMEM((K, W), jnp.float32),
        pltpu.VMEM((BZ, W), jnp.float32),
        pltpu.VMEM_SHARED((TP, W), jnp.float32),
        pltpu.SemaphoreType.DMA,
        pltpu.SemaphoreType.DMA,
    ]

    def body(*refs):
        if gather:
            (src_hbm, ia_hbm, ib_hbm, out_hbm,
             ia_v, ib_v, r0, r1, bounce, shared, g0, g1) = refs
        else:
            (ib_hbm, out_hbm,
             ib_v, r0, r1, bounce, shared, g0, g1) = refs
        c = lax.axis_index("c")
        s = lax.axis_index("s")
        wid = c * NS + s

        z16 = jnp.zeros((16,), jnp.float32)

        def zrow(i, carry):
            for w in range(W16):
                bounce[i, pl.ds(w * 16, 16)] = z16
            return carry

        lax.fori_loop(0, BZ, zrow, 0)
        for m in range(4):
            pltpu.sync_copy(bounce, shared.at[pl.ds(s * TPS + m * BZ, BZ)])

        pltpu.sync_copy(ib_hbm.at[wid], ib_v)
        if gather:
            pltpu.sync_copy(ia_hbm.at[wid], ia_v)
        else:
            o16 = jnp.ones((16,), jnp.float32)

            def orow(i, carry):
                for w in range(W16):
                    r0[i, pl.ds(w * 16, 16)] = o16
                return carry

            lax.fori_loop(0, K, orow, 0)

        plsc.subcore_barrier()

        if gather:
            pltpu.async_copy(src_hbm.at[ia_v.at[0]], r0, g0)
            pltpu.async_copy(src_hbm.at[ia_v.at[1]], r1, g1)

            def pair(j2, carry):
                j = j2 * 2
                pltpu.make_async_copy(src_hbm.at[ia_v.at[0]], r0, g0).wait()
                pltpu.sync_copy(r0, shared.at[ib_v.at[j]], add=True)

                @pl.when(j + 2 < NCH)
                def _():
                    pltpu.async_copy(src_hbm.at[ia_v.at[j + 2]], r0, g0)

                pltpu.make_async_copy(src_hbm.at[ia_v.at[1]], r1, g1).wait()
                pltpu.sync_copy(r1, shared.at[ib_v.at[j + 1]], add=True)

                @pl.when(j + 3 < NCH)
                def _():
                    pltpu.async_copy(src_hbm.at[ia_v.at[j + 3]], r1, g1)

                return carry

            lax.fori_loop(0, NCH // 2, pair, 0)
            if NCH % 2 == 1:
                pltpu.make_async_copy(src_hbm.at[ia_v.at[0]], r0, g0).wait()
                pltpu.sync_copy(r0, shared.at[ib_v.at[NCH - 1]], add=True)
        else:
            def ch(j, carry):
                pltpu.sync_copy(r0, shared.at[ib_v.at[j]], add=True)
                return carry

            lax.fori_loop(0, NCH, ch, 0)

        plsc.subcore_barrier()
        for m in range(4):
            pltpu.sync_copy(shared.at[pl.ds(s * TPS + m * BZ, BZ)], bounce)
            pltpu.sync_copy(bounce, out_hbm.at[c, pl.ds(s * TPS + m * BZ, BZ)])

    return pl.kernel(
        body,
        out_type=jax.ShapeDtypeStruct((NC, TP, W), jnp.float32),
        mesh=mesh,
        scratch_types=scratch,
        compiler_params=pltpu.CompilerParams(use_tc_tiling_on_sc=False),
    )


def _softplus(x):
    return jnp.maximum(x, 0.0) + jnp.log(1.0 + jnp.exp(-jnp.abs(x)))


TPC = -(-C // 2048) * 2048
TPL = -(-2 * V // 2048) * 2048


def _part(b, w, core, off=0):
    return pl.BlockSpec((1, b, w), lambda i, c=core, o=off: (c, i + o, 0))


def _full(shape):
    return pl.BlockSpec(shape, lambda i: (0, 0))


def _rows(b, w):
    return pl.BlockSpec((b, w), lambda i: (i, 0))


def _query_mlp(v, noise, w1v, w1n, b1, w2, b2):
    BV = 1000

    def body(v_r, n_r, w1v_r, w1n_r, b1_r, w2_r, b2_r, q_r, t1_r, t2_r):
        i = pl.program_id(0)
        u = jnp.maximum(v_r[...] @ w1v_r[...] + n_r[...] @ w1n_r[...]
                        + b1_r[...], 0.0)
        q = u @ w2_r[...] + b2_r[...]
        sp = _softplus(q)
        tab = jnp.where(i >= V // BV, sp - q, sp)
        q_r[...] = q
        t1_r[...] = tab[:, :16]
        t2_r[...] = tab[:, 16:]

    half = lambda i: (i % (V // BV), 0)
    return pl.pallas_call(
        body,
        grid=(2 * V // BV,),
        in_specs=[pl.BlockSpec((BV, 128), half), pl.BlockSpec((BV, 4), half),
                  _full((128, 38)), _full((4, 38)), _full((1, 38)),
                  _full((38, 32)), _full((1, 32))],
        out_specs=[pl.BlockSpec((BV, 32), half),
                   _rows(BV, 16), _rows(BV, 16)],
        out_shape=[jax.ShapeDtypeStruct((V, 32), jnp.float32),
                   jax.ShapeDtypeStruct((2 * V, 16), jnp.float32),
                   jax.ShapeDtypeStruct((2 * V, 16), jnp.float32)],
    )(v, noise, w1v, w1n, b1, w2, b2)


def _clause_a(cs, cvh1, cvh2, gc, w1, b1, w2, b2):
    BC = 1000

    def body(cs_r, h1a_r, h1b_r, h2a_r, h2b_r, g_r, w1_r, b1_r, w2_r, b2_r,
             e1_r, e2_r, v1_r, v2_r, x_r, s1_r, s2_r):
        i = pl.program_id(0)
        cv = jnp.concatenate([h1a_r[0] + h1b_r[0], h2a_r[0] + h2b_r[0]],
                             axis=1)
        expcv = jnp.exp(-cv)
        u = jnp.concatenate([cs_r[...], 4.0 * expcv], axis=1)
        h = jnp.maximum(u @ w1_r[...] + b1_r[...], 0.0)
        dd = h @ w2_r[...] + b2_r[...]
        vla = dd[:, :Q]
        x = dd[:, Q:]
        e1_r[...] = expcv[:, :16]
        e2_r[...] = expcv[:, 16:]
        v1_r[...] = vla[:, :16]
        v2_r[...] = vla[:, 16:]
        x_r[...] = x
        m2 = jnp.broadcast_to(jnp.mean(x * x, axis=1, keepdims=True), (BC, 8))
        g = g_r[...]
        ds1 = lax.dot_general(g, x, (((0,), (0,)), ((), ())),
                              preferred_element_type=jnp.float32)
        ds2 = lax.dot_general(g, m2, (((0,), (0,)), ((), ())),
                              preferred_element_type=jnp.float32)

        @pl.when(i == 0)
        def _():
            s1_r[...] = jnp.zeros_like(s1_r)
            s2_r[...] = jnp.zeros_like(s2_r)

        s1_r[...] += ds1
        s2_r[...] += ds2

    return pl.pallas_call(
        body,
        grid=(C // BC,),
        in_specs=[_rows(BC, 128),
                  _part(BC, 16, 0), _part(BC, 16, 1),
                  _part(BC, 16, 0), _part(BC, 16, 1),
                  _rows(BC, 32),
                  _full((160, 204)), _full((1, 204)),
                  _full((204, 160)), _full((1, 160))],
        out_specs=[_rows(BC, 16), _rows(BC, 16), _rows(BC, 16),
                   _rows(BC, 16), _rows(BC, 128),
                   _full((G, 128)), _full((G, 8))],
        out_shape=[jax.ShapeDtypeStruct((C, 16), jnp.float32),
                   jax.ShapeDtypeStruct((C, 16), jnp.float32),
                   jax.ShapeDtypeStruct((C, 16), jnp.float32),
                   jax.ShapeDtypeStruct((C, 16), jnp.float32),
                   jax.ShapeDtypeStruct((C, 128), jnp.float32),
                   jax.ShapeDtypeStruct((G, 128), jnp.float32),
                   jax.ShapeDtypeStruct((G, 8), jnp.float32)],
    )(cs, cvh1, cvh1, cvh2, cvh2, gc, w1, b1, w2, b2)


def _pairnorm_body(x, g, s1, s2, cnt):
    c1 = cnt[:, :1]
    mean = s1 / c1
    m2g = s2[:, :1] / c1
    var = m2g - jnp.mean(mean * mean, axis=1, keepdims=True)
    scale = lax.rsqrt(var + 1e-6)
    meanb = g @ mean
    scaleb = g @ scale
    return (x - meanb) * scaleb


def _clause_b(x, cs_old, gc, s1, s2, cnt):
    BC = 1000

    def body(x_r, cs_r, g_r, s1_r, s2_r, cnt_r, out_r):
        pn = _pairnorm_body(x_r[...], g_r[...], s1_r[...], s2_r[...],
                            cnt_r[...])
        out_r[...] = pn + 0.1 * cs_r[...]

    return pl.pallas_call(
        body,
        grid=(C // BC,),
        in_specs=[_rows(BC, 128), _rows(BC, 128), _rows(BC, 32),
                  _full((G, 128)), _full((G, 8)), _full((G, 8))],
        out_specs=[_rows(BC, 128)],
        out_shape=[jax.ShapeDtypeStruct((C, 128), jnp.float32)],
    )(x, cs_old, gc, s1, s2, cnt)[0]


def _update_a(q, teh1, teh2, tvh1, tvh2, v, dw, gv,
              w1, b1, w2, b2, w3, b3):
    BV = 1000
    NB = V // BV

    def body(q_r, e1pa, e1pb, e2pa, e2pb, e1na, e1nb, e2na, e2nb,
             v1pa, v1pb, v2pa, v2pb, v1na, v1nb, v2na, v2nb,
             v_r, dw_r, g_r,
             w1_r, b1_r, w2_r, b2_r, w3_r, b3_r, x_r, s1_r, s2_r):
        i = pl.program_id(0)
        qv = q_r[...]
        sig = 1.0 / (1.0 + jnp.exp(-qv))
        rs = lambda a, b: a[0] + b[0]
        te_p = jnp.concatenate([rs(e1pa, e1pb), rs(e2pa, e2pb)], axis=1)
        te_n = jnp.concatenate([rs(e1na, e1nb), rs(e2na, e2nb)], axis=1)
        tv_p = jnp.concatenate([rs(v1pa, v1pb), rs(v2pa, v2pb)], axis=1)
        tv_n = jnp.concatenate([rs(v1na, v1nb), rs(v2na, v2nb)], axis=1)
        dwv = dw_r[...]
        grad = (-sig * te_p + (1.0 - sig) * te_n) * dwv[:, :1]
        lp = tv_p * dwv[:, 8:9]
        ln = tv_n * dwv[:, 16:17]
        u = jnp.concatenate([grad, v_r[...], lp, ln], axis=1)
        h1 = jnp.maximum(u @ w1_r[...] + b1_r[...], 0.0)
        h2 = jnp.maximum(h1 @ w2_r[...] + b2_r[...], 0.0)
        x = h2 @ w3_r[...] + b3_r[...]
        x_r[...] = x
        m2 = jnp.broadcast_to(jnp.mean(x * x, axis=1, keepdims=True), (BV, 8))
        g = g_r[...]
        ds1 = lax.dot_general(g, x, (((0,), (0,)), ((), ())),
                              preferred_element_type=jnp.float32)
        ds2 = lax.dot_general(g, m2, (((0,), (0,)), ((), ())),
                              preferred_element_type=jnp.float32)

        @pl.when(i == 0)
        def _():
            s1_r[...] = jnp.zeros_like(s1_r)
            s2_r[...] = jnp.zeros_like(s2_r)

        s1_r[...] += ds1
        s2_r[...] += ds2

    return pl.pallas_call(
        body,
        grid=(V // BV,),
        in_specs=[_rows(BV, 32),
                  _part(BV, 16, 0), _part(BV, 16, 1),
                  _part(BV, 16, 0), _part(BV, 16, 1),
                  _part(BV, 16, 0, NB), _part(BV, 16, 1, NB),
                  _part(BV, 16, 0, NB), _part(BV, 16, 1, NB),
                  _part(BV, 16, 0), _part(BV, 16, 1),
                  _part(BV, 16, 0), _part(BV, 16, 1),
                  _part(BV, 16, 0, NB), _part(BV, 16, 1, NB),
                  _part(BV, 16, 0, NB), _part(BV, 16, 1, NB),
                  _rows(BV, 128), _rows(BV, 24), _rows(BV, 32),
                  _full((224, 230)), _full((1, 230)),
                  _full((230, 230)), _full((1, 230)),
                  _full((230, 128)), _full((1, 128))],
        out_specs=[_rows(BV, 128), _full((G, 128)), _full((G, 8))],
        out_shape=[jax.ShapeDtypeStruct((V, 128), jnp.float32),
                   jax.ShapeDtypeStruct((G, 128), jnp.float32),
                   jax.ShapeDtypeStruct((G, 8), jnp.float32)],
    )(q, teh1, teh1, teh2, teh2, teh1, teh1, teh2, teh2,
      tvh1, tvh1, tvh2, tvh2, tvh1, tvh1, tvh2, tvh2,
      v, dw, gv, w1, b1, w2, b2, w3, b3)


def _update_b(x, v_old, gv, s1, s2, cnt, wo1, bo1, wo2, bo2):
    BV = 1000

    def body(x_r, v_r, g_r, s1_r, s2_r, cnt_r, wo1_r, bo1_r, wo2_r, bo2_r,
             vn_r, tab_r):
        i = pl.program_id(0)
        pn = _pairnorm_body(x_r[...], g_r[...], s1_r[...], s2_r[...],
                            cnt_r[...])
        vnew = pn + 0.1 * v_r[...]
        vn_r[...] = vnew
        h = jnp.maximum(vnew @ wo1_r[...] + bo1_r[...], 0.0)
        lg = h @ wo2_r[...] + bo2_r[...]
        sp = _softplus(lg)
        hard = (lg > 0).astype(jnp.float32)
        neg = i >= V // BV
        tab_r[...] = jnp.where(neg,
                               jnp.concatenate([sp - lg, 1.0 - hard], axis=1),
                               jnp.concatenate([sp, hard], axis=1))

    half = lambda i: (i % (V // BV), 0)
    return pl.pallas_call(
        body,
        grid=(2 * V // BV,),
        in_specs=[pl.BlockSpec((BV, 128), half), pl.BlockSpec((BV, 128), half),
                  pl.BlockSpec((BV, 32), half),
                  _full((G, 128)), _full((G, 8)), _full((G, 8)),
                  _full((128, 128)), _full((1, 128)),
                  _full((128, 8)), _full((1, 8))],
        out_specs=[pl.BlockSpec((BV, 128), half), _rows(BV, 16)],
        out_shape=[jax.ShapeDtypeStruct((V, 128), jnp.float32),
                   jax.ShapeDtypeStruct((2 * V, 16), jnp.float32)],
    )(x, v_old, gv, s1, s2, cnt, wo1, bo1, wo2, bo2)


def _loss_reduce(accp, gc):
    BC = 1000

    def body(aa_r, ab_r, g_r, pg_r):
        i = pl.program_id(0)
        a = aa_r[0] + ab_r[0]
        cv8 = jnp.exp(-a[:, :LM])
        pcl = cv8 * (-jnp.log(1.0 - cv8 + 1e-6))
        hard = jnp.minimum(a[:, LM:], 1.0)
        both = jnp.concatenate([pcl, hard], axis=1)
        d = lax.dot_general(g_r[...], both, (((0,), (0,)), ((), ())),
                            preferred_element_type=jnp.float32)

        @pl.when(i == 0)
        def _():
            pg_r[...] = jnp.zeros_like(pg_r)

        pg_r[...] += d

    return pl.pallas_call(
        body,
        grid=(C // BC,),
        in_specs=[_part(BC, 16, 0), _part(BC, 16, 1), _rows(BC, 32)],
        out_specs=[_full((G, 16))],
        out_shape=[jax.ShapeDtypeStruct((G, 16), jnp.float32)],
    )(accp, accp, gc)[0]


def kernel(params, variable_indices, clause_indices, polarity,
           variable_batch, clause_batch):
    (wq1, bq1), (wq2, bq2) = params['variables_query']
    (wc1, bc1), (wc2, bc2) = params['clause_mlp']
    (wu1, bu1), (wu2, bu2), (wu3, bu3) = params['update_gate']
    (wo1, bo1), (wo2, bo2) = params['variables_output']
    bq1 = bq1.reshape(1, -1); bq2 = bq2.reshape(1, -1)
    bc1 = bc1.reshape(1, -1); bc2 = bc2.reshape(1, -1)
    bu1 = bu1.reshape(1, -1); bu2 = bu2.reshape(1, -1)
    bu3 = bu3.reshape(1, -1)
    bo1 = bo1.reshape(1, -1); bo2 = bo2.reshape(1, -1)

    vi = variable_indices.astype(jnp.int32)
    ci = clause_indices.astype(jnp.int32)
    lit = vi + jnp.where(polarity > 0, 0, V).astype(jnp.int32)
    lit2 = lit.reshape(NW, NCH, K)
    ci2 = ci.reshape(NW, NCH, K)

    gv = (variable_batch[:, None] == jnp.arange(G)[None, :]).astype(jnp.float32)
    gc = (clause_batch[:, None] == jnp.arange(G)[None, :]).astype(jnp.float32)
    var_counts = jnp.maximum(jnp.sum(gv, axis=0), 1.0)[:, None]
    cl_counts = jnp.maximum(jnp.sum(gc, axis=0), 1.0)[:, None]
    cnt_v8 = jnp.broadcast_to(var_counts, (G, 8))
    cnt_c8 = jnp.broadcast_to(cl_counts, (G, 8))

    edge_lc16 = _make_edge_op(2 * V, C, 16)
    edge_cl16 = _make_edge_op(C, 2 * V, 16)

    ones_c16 = jnp.ones((C, 16), jnp.float32)
    degp = edge_cl16(ones_c16, ci2, lit2)
    deg = (degp[0] + degp[1])[:2 * V, :1]
    degree_weight = lax.rsqrt(jnp.maximum(deg, 1.0))
    vdw = 4.0 * lax.rsqrt(jnp.maximum(deg[:V] + deg[V], 1.0))
    dw = jnp.concatenate([jnp.broadcast_to(vdw, (V, 8)),
                          jnp.broadcast_to(degree_weight[:V], (V, 8)),
                          jnp.broadcast_to(degree_weight[V:], (V, 8))],
                         axis=1)

    variables = jnp.ones((V, D), jnp.float32)
    clause_state = jnp.ones((C, D), jnp.float32)
    noise_key = jax.random.key(42)
    costs = jnp.square(jnp.arange(1, LM + 1, dtype=jnp.float32))

    wq1v = wq1[:D]
    wq1n = wq1[D:]

    losses = []
    solveds = []
    for step in range(STEPS):
        noise = jax.random.normal(jax.random.fold_in(noise_key, step),
                                  (V, 4), jnp.float32)
        q, lt1, lt2 = _query_mlp(variables, noise, wq1v, wq1n, bq1, wq2, bq2)

        cvh1 = edge_lc16(lt1, lit2, ci2)
        cvh2 = edge_lc16(lt2, lit2, ci2)

        e1, e2, v1, v2, x_c, s1c, s2c = _clause_a(clause_state, cvh1, cvh2,
                                                  gc, wc1, bc1, wc2, bc2)
        clause_state = _clause_b(x_c, clause_state, gc, s1c, s2c, cnt_c8)

        teh1 = edge_cl16(e1, ci2, lit2)
        teh2 = edge_cl16(e2, ci2, lit2)
        tvh1 = edge_cl16(v1, ci2, lit2)
        tvh2 = edge_cl16(v2, ci2, lit2)

        x_v, s1v, s2v = _update_a(q, teh1, teh2, tvh1, tvh2,
                                  variables, dw, gv,
                                  wu1, bu1, wu2, bu2, wu3, bu3)
        variables, tab16 = _update_b(x_v, variables, gv, s1v, s2v,
                                     cnt_v8, wo1, bo1, wo2, bo2)

        accp = edge_lc16(tab16, lit2, ci2)

        pg = _loss_reduce(accp, gc)
        pgl = jnp.sqrt(pg[:, :LM] + 1e-6) - np.sqrt(1e-6)
        sorted_loss = -jnp.sort(-pgl, axis=-1)
        losses.append(jnp.sum(sorted_loss * costs) / jnp.sum(costs))
        gh = (pg[:, LM:] / cl_counts) > (1.0 - 1e-6)
        solveds.append(jnp.any(gh, axis=1))

    loss = jnp.sum(jnp.stack(losses)) / STEPS
    return loss, jnp.stack(solveds)
